# Initial kernel scaffold; baseline (speedup 1.0000x reference)
import functools

import jax
import jax.numpy as jnp
from jax import lax
from jax.experimental import pallas as pl
from jax.experimental.pallas import tpu as pltpu
from jax.experimental.pallas import tpu_sc as plsc

N_NODES = 10000
N_EDGES = 320000
N_PAIR = 20000
D = 128

NC = 2
NS = 16
NW = NC * NS

EW = N_EDGES // NW
CH = 80
NCHUNK = EW // CH
RPT = N_NODES // NS

_sc_mesh = plsc.VectorSubcoreMesh(core_axis_name="c", subcore_axis_name="s")


def _seg_sum_body(x_hbm, src_hbm, dst_hbm, zfeat_hbm, zdeg_hbm, ones_hbm,
                  agg_hbm, deg_hbm,
                  src_v, dst_v, rows_v, ones_v, acc_s, dacc_s, sem):
    c = lax.axis_index("c")
    s = lax.axis_index("s")
    w = c * NS + s
    rbase = s * RPT
    pltpu.sync_copy(zfeat_hbm, acc_s.at[pl.ds(rbase, RPT)])
    pltpu.sync_copy(zdeg_hbm, dacc_s.at[pl.ds(rbase, RPT)])
    pltpu.sync_copy(ones_hbm, ones_v)
    plsc.subcore_barrier()
    ebase = w * EW

    def body(k, carry):
        off = ebase + k * CH
        pltpu.sync_copy(src_hbm.at[pl.ds(off, CH)], src_v)
        pltpu.sync_copy(dst_hbm.at[pl.ds(off, CH)], dst_v)
        pltpu.async_copy(x_hbm.at[src_v], rows_v, sem).wait()
        pltpu.sync_copy(rows_v, acc_s.at[dst_v], add=True)
        pltpu.sync_copy(ones_v, dacc_s.at[dst_v], add=True)
        return carry

    lax.fori_loop(0, NCHUNK, body, 0)
    plsc.subcore_barrier()
    pltpu.sync_copy(acc_s.at[pl.ds(rbase, RPT)], agg_hbm.at[c, pl.ds(rbase, RPT)])
    pltpu.sync_copy(dacc_s.at[pl.ds(rbase, RPT)], deg_hbm.at[c, pl.ds(rbase, RPT)])


_seg_sum = pl.kernel(
    _seg_sum_body,
    out_type=[jax.ShapeDtypeStruct((NC, N_NODES, D), jnp.float32),
              jax.ShapeDtypeStruct((NC, N_NODES, 16), jnp.float32)],
    mesh=_sc_mesh,
    scratch_types=[pltpu.VMEM((CH,), jnp.int32),
                   pltpu.VMEM((CH,), jnp.int32),
                   pltpu.VMEM((CH, D), jnp.float32),
                   pltpu.VMEM((CH, 16), jnp.float32),
                   pltpu.VMEM_SHARED((N_NODES, D), jnp.float32),
                   pltpu.VMEM_SHARED((N_NODES, 16), jnp.float32),
                   pltpu.SemaphoreType.DMA],
)

PCH = 80
NPCHUNK = N_PAIR // PCH
CPW = (NPCHUNK + NW - 1) // NW


def _pair_gather_body(h_hbm, ps_hbm, pd_hbm, ns_hbm, nd_hbm,
                      a_hbm, b_hbm,
                      ia_v, ib_v, a_v, b_v, sem):
    c = lax.axis_index("c")
    s = lax.axis_index("s")
    w = c * NS + s

    def one_set(src_idx, dst_idx, out_base):
        def body(i, carry):
            k = w + i * NW

            @pl.when(k < NPCHUNK)
            def _():
                off = k * PCH
                pltpu.sync_copy(src_idx.at[pl.ds(off, PCH)], ia_v)
                pltpu.sync_copy(dst_idx.at[pl.ds(off, PCH)], ib_v)
                ca = pltpu.async_copy(h_hbm.at[ia_v], a_v, sem)
                cb = pltpu.async_copy(h_hbm.at[ib_v], b_v, sem)
                ca.wait()
                cb.wait()
                pltpu.sync_copy(a_v, a_hbm.at[pl.ds(out_base + off, PCH)])
                pltpu.sync_copy(b_v, b_hbm.at[pl.ds(out_base + off, PCH)])

            return carry

        lax.fori_loop(0, CPW, body, 0)

    one_set(ps_hbm, pd_hbm, 0)
    one_set(ns_hbm, nd_hbm, N_PAIR)


_pair_gather = pl.kernel(
    _pair_gather_body,
    out_type=[jax.ShapeDtypeStruct((2 * N_PAIR, D), jnp.float32),
              jax.ShapeDtypeStruct((2 * N_PAIR, D), jnp.float32)],
    mesh=_sc_mesh,
    scratch_types=[pltpu.VMEM((PCH,), jnp.int32),
                   pltpu.VMEM((PCH,), jnp.int32),
                   pltpu.VMEM((PCH, D), jnp.float32),
                   pltpu.VMEM((PCH, D), jnp.float32),
                   pltpu.SemaphoreType.DMA],
)

BM = 2000


def _combine_body(x_ref, aggp_ref, degp_ref, ws_ref, wn_ref, b_ref, o_ref, *, relu):
    agg = aggp_ref[0] + aggp_ref[1]
    deg = degp_ref[0, :, 0:1] + degp_ref[1, :, 0:1]
    hn = agg / jnp.maximum(deg, 1.0)
    acc = jnp.dot(x_ref[...], ws_ref[...], preferred_element_type=jnp.float32)
    acc = acc + jnp.dot(hn, wn_ref[...], preferred_element_type=jnp.float32)
    acc = acc + b_ref[...]
    if relu:
        acc = jnp.maximum(acc, 0.0)
    o_ref[...] = acc


def _make_combine(relu):
    return pl.pallas_call(
        functools.partial(_combine_body, relu=relu),
        grid=(N_NODES // BM,),
        in_specs=[
            pl.BlockSpec((BM, D), lambda i: (i, 0)),
            pl.BlockSpec((NC, BM, D), lambda i: (0, i, 0)),
            pl.BlockSpec((NC, BM, 16), lambda i: (0, i, 0)),
            pl.BlockSpec((D, D), lambda i: (0, 0)),
            pl.BlockSpec((D, D), lambda i: (0, 0)),
            pl.BlockSpec((1, D), lambda i: (0, 0)),
        ],
        out_specs=pl.BlockSpec((BM, D), lambda i: (i, 0)),
        out_shape=jax.ShapeDtypeStruct((N_NODES, D), jnp.float32),
    )


_combine_relu = _make_combine(True)
_combine_lin = _make_combine(False)

BP = 4000


def _mlp_body(a_ref, b_ref, w1_ref, b1_ref, w2_ref, b2_ref, w3_ref, b3_ref, o_ref):
    e = a_ref[...] * b_ref[...]
    t = jnp.dot(e, w1_ref[...], preferred_element_type=jnp.float32) + b1_ref[...]
    t = jnp.maximum(t, 0.0)
    t = jnp.dot(t, w2_ref[...], preferred_element_type=jnp.float32) + b2_ref[...]
    t = jnp.maximum(t, 0.0)
    o_ref[...] = jnp.dot(t, w3_ref[...], preferred_element_type=jnp.float32) + b3_ref[...]


_mlp = pl.pallas_call(
    _mlp_body,
    grid=(2 * N_PAIR // BP,),
    in_specs=[
        pl.BlockSpec((BP, D), lambda i: (i, 0)),
        pl.BlockSpec((BP, D), lambda i: (i, 0)),
        pl.BlockSpec((D, D), lambda i: (0, 0)),
        pl.BlockSpec((1, D), lambda i: (0, 0)),
        pl.BlockSpec((D, D), lambda i: (0, 0)),
        pl.BlockSpec((1, D), lambda i: (0, 0)),
        pl.BlockSpec((D, 1), lambda i: (0, 0)),
        pl.BlockSpec((1, 1), lambda i: (0, 0)),
    ],
    out_specs=pl.BlockSpec((BP, 1), lambda i: (i, 0)),
    out_shape=jax.ShapeDtypeStruct((2 * N_PAIR, 1), jnp.float32),
)


def kernel(x, edge_index_l0, edge_index_l1, pos_edge_index, neg_edge_index,
           Wself0, Wneigh0, b0, Wself1, Wneigh1, b1,
           Wd1, bd1, Wd2, bd2, Wd3, bd3):
    zfeat = jnp.zeros((RPT, D), jnp.float32)
    zdeg = jnp.zeros((RPT, 16), jnp.float32)
    ones = jnp.ones((CH, 16), jnp.float32)

    aggp0, degp0 = _seg_sum(x, edge_index_l0[0], edge_index_l0[1],
                            zfeat, zdeg, ones)
    h1 = _combine_relu(x, aggp0, degp0, Wself0.T, Wneigh0.T, b0.reshape(1, D))
    aggp1, degp1 = _seg_sum(h1, edge_index_l1[0], edge_index_l1[1],
                            zfeat, zdeg, ones)
    h2 = _combine_lin(h1, aggp1, degp1, Wself1.T, Wneigh1.T, b1.reshape(1, D))
    a, b = _pair_gather(h2, pos_edge_index[0], pos_edge_index[1],
                        neg_edge_index[0], neg_edge_index[1])
    out = _mlp(a, b, Wd1.T, bd1.reshape(1, D), Wd2.T, bd2.reshape(1, D),
               Wd3.T, bd3.reshape(1, 1))
    return out[:N_PAIR], out[N_PAIR:]


# speedup vs baseline: 10.6422x; 10.6422x over previous
import functools

import jax
import jax.numpy as jnp
from jax import lax
from jax.experimental import pallas as pl
from jax.experimental.pallas import tpu as pltpu
from jax.experimental.pallas import tpu_sc as plsc

N_NODES = 10000
N_EDGES = 320000
N_PAIR = 20000
D = 128

NC = 2
NS = 16
NW = NC * NS

EW = N_EDGES // NW
CH = 80
NCHUNK = EW // CH
N_PAD = 10240
RPT = N_PAD // NS

_sc_mesh = plsc.VectorSubcoreMesh(core_axis_name="c", subcore_axis_name="s")


NBUF = 3
SB = 25
NSB = NCHUNK // SB


def _seg_sum_body(x_hbm, src_hbm, dst_hbm, zfeat_hbm, zdeg_hbm,
                  agg_hbm, degw_hbm,
                  srcb_v, dstb_v,
                  rows_v0, rows_v1, rows_v2,
                  dloc_v, acc_s, sem0, sem1, sem2):
    c = lax.axis_index("c")
    s = lax.axis_index("s")
    w = c * NS + s
    rbase = s * RPT
    rows = (rows_v0, rows_v1, rows_v2)
    sems = (sem0, sem1, sem2)
    ones16 = jnp.ones((16,), jnp.float32)

    pltpu.sync_copy(zfeat_hbm, acc_s.at[pl.ds(rbase, RPT)])
    pltpu.sync_copy(zdeg_hbm, dloc_v)
    plsc.subcore_barrier()

    def sblock(g, carry):
        pltpu.sync_copy(src_hbm.at[w, g], srcb_v)
        pltpu.sync_copy(dst_hbm.at[w, g], dstb_v)
        for b in range(NBUF):
            pltpu.async_copy(x_hbm.at[srcb_v.at[b]], rows[b], sems[b])

        def body(gg, carry2):
            for b in range(NBUF):
                kk = gg * NBUF + b

                @pl.when(kk < SB)
                def _():
                    pltpu.make_async_copy(x_hbm.at[srcb_v.at[kk]],
                                          rows[b], sems[b]).wait()
                    for j in range(CH // 16):
                        idx = dstb_v[kk, pl.ds(j * 16, 16)]
                        plsc.addupdate_scatter(dloc_v, [idx], ones16)
                    pltpu.sync_copy(rows[b], acc_s.at[dstb_v.at[kk]], add=True)
                    kn = kk + NBUF

                    @pl.when(kn < SB)
                    def __():
                        pltpu.async_copy(x_hbm.at[srcb_v.at[kn]], rows[b], sems[b])

            return carry2

        lax.fori_loop(0, (SB + NBUF - 1) // NBUF, body, 0)
        return carry

    lax.fori_loop(0, NSB, sblock, 0)
    plsc.subcore_barrier()
    pltpu.sync_copy(acc_s.at[pl.ds(rbase, RPT)], agg_hbm.at[c, pl.ds(rbase, RPT)])
    pltpu.sync_copy(dloc_v, degw_hbm.at[w])


_seg_sum = pl.kernel(
    _seg_sum_body,
    out_type=[jax.ShapeDtypeStruct((NC, N_PAD, D), jnp.float32),
              jax.ShapeDtypeStruct((NW, N_PAD), jnp.float32)],
    mesh=_sc_mesh,
    scratch_types=([pltpu.VMEM((SB, CH), jnp.int32)] * 2
                   + [pltpu.VMEM((CH, D), jnp.float32)] * NBUF
                   + [pltpu.VMEM((N_PAD,), jnp.float32),
                      pltpu.VMEM_SHARED((N_PAD, D), jnp.float32)]
                   + [pltpu.SemaphoreType.DMA] * NBUF),
    compiler_params=pltpu.CompilerParams(needs_layout_passes=False),
)

PCH = 80
NPCHUNK = N_PAIR // PCH
CPW = 8
NPPAD = CPW * NW


def _pair_gather_body(h_hbm, ps_hbm, pd_hbm, ns_hbm, nd_hbm,
                      a_hbm, b_hbm,
                      ia_v, ib_v,
                      a0_v, a1_v, b0_v, b1_v, semA0, semA1, semB0, semB1):
    c = lax.axis_index("c")
    s = lax.axis_index("s")
    w = c * NS + s
    avs = (a0_v, a1_v)
    bvs = (b0_v, b1_v)
    semsA = (semA0, semA1)
    semsB = (semB0, semB1)

    for eset, (src_idx, dst_idx) in enumerate(((ps_hbm, pd_hbm), (ns_hbm, nd_hbm))):
        out_base = eset * N_PAIR
        pltpu.sync_copy(src_idx.at[pl.ds(w * CPW, CPW)], ia_v)
        pltpu.sync_copy(dst_idx.at[pl.ds(w * CPW, CPW)], ib_v)

        def issue(i, b):
            k = w * CPW + i

            @pl.when(k < NPCHUNK)
            def _():
                pltpu.async_copy(h_hbm.at[ia_v.at[i]], avs[b], semsA[b])
                pltpu.async_copy(h_hbm.at[ib_v.at[i]], bvs[b], semsB[b])

        for b in range(2):
            issue(b, b)
        for i in range(CPW):
            b = i & 1
            k = w * CPW + i

            @pl.when(k < NPCHUNK)
            def _():
                off = k * PCH
                pltpu.make_async_copy(h_hbm.at[ia_v.at[i]], avs[b], semsA[b]).wait()
                pltpu.make_async_copy(h_hbm.at[ib_v.at[i]], bvs[b], semsB[b]).wait()
                pltpu.sync_copy(avs[b], a_hbm.at[pl.ds(out_base + off, PCH)])
                pltpu.sync_copy(bvs[b], b_hbm.at[pl.ds(out_base + off, PCH)])

            if i + 2 < CPW:
                issue(i + 2, b)


_pair_gather = pl.kernel(
    _pair_gather_body,
    out_type=[jax.ShapeDtypeStruct((2 * N_PAIR, D), jnp.float32),
              jax.ShapeDtypeStruct((2 * N_PAIR, D), jnp.float32)],
    mesh=_sc_mesh,
    scratch_types=([pltpu.VMEM((CPW, PCH), jnp.int32)] * 2
                   + [pltpu.VMEM((PCH, D), jnp.float32)] * 4
                   + [pltpu.SemaphoreType.DMA] * 4),
)

BM = 2048


def _combine_body(x_ref, aggp_ref, degp_ref, ws_ref, wn_ref, b_ref, o_ref, *, relu):
    agg = aggp_ref[0] + aggp_ref[1]
    deg = jnp.sum(degp_ref[...], axis=0)[:, None]
    hn = agg / jnp.maximum(deg, 1.0)
    acc = jnp.dot(x_ref[...], ws_ref[...], preferred_element_type=jnp.float32)
    acc = acc + jnp.dot(hn, wn_ref[...], preferred_element_type=jnp.float32)
    acc = acc + b_ref[...]
    if relu:
        acc = jnp.maximum(acc, 0.0)
    o_ref[...] = acc


def _make_combine(relu):
    return pl.pallas_call(
        functools.partial(_combine_body, relu=relu),
        grid=(N_PAD // BM,),
        in_specs=[
            pl.BlockSpec((BM, D), lambda i: (i, 0)),
            pl.BlockSpec((NC, BM, D), lambda i: (0, i, 0)),
            pl.BlockSpec((NW, BM), lambda i: (0, i)),
            pl.BlockSpec((D, D), lambda i: (0, 0)),
            pl.BlockSpec((D, D), lambda i: (0, 0)),
            pl.BlockSpec((1, D), lambda i: (0, 0)),
        ],
        out_specs=pl.BlockSpec((BM, D), lambda i: (i, 0)),
        out_shape=jax.ShapeDtypeStruct((N_NODES, D), jnp.float32),
    )


_combine_relu = _make_combine(True)
_combine_lin = _make_combine(False)

BP = 4000


def _mlp_body(a_ref, b_ref, w1_ref, b1_ref, w2_ref, b2_ref, w3_ref, b3_ref, o_ref):
    e = a_ref[...] * b_ref[...]
    t = jnp.dot(e, w1_ref[...], preferred_element_type=jnp.float32) + b1_ref[...]
    t = jnp.maximum(t, 0.0)
    t = jnp.dot(t, w2_ref[...], preferred_element_type=jnp.float32) + b2_ref[...]
    t = jnp.maximum(t, 0.0)
    o_ref[...] = jnp.dot(t, w3_ref[...], preferred_element_type=jnp.float32) + b3_ref[...]


_mlp = pl.pallas_call(
    _mlp_body,
    grid=(2 * N_PAIR // BP,),
    in_specs=[
        pl.BlockSpec((BP, D), lambda i: (i, 0)),
        pl.BlockSpec((BP, D), lambda i: (i, 0)),
        pl.BlockSpec((D, D), lambda i: (0, 0)),
        pl.BlockSpec((1, D), lambda i: (0, 0)),
        pl.BlockSpec((D, D), lambda i: (0, 0)),
        pl.BlockSpec((1, D), lambda i: (0, 0)),
        pl.BlockSpec((D, 1), lambda i: (0, 0)),
        pl.BlockSpec((1, 1), lambda i: (0, 0)),
    ],
    out_specs=pl.BlockSpec((BP, 1), lambda i: (i, 0)),
    out_shape=jax.ShapeDtypeStruct((2 * N_PAIR, 1), jnp.float32),
)


def kernel(x, edge_index_l0, edge_index_l1, pos_edge_index, neg_edge_index,
           Wself0, Wneigh0, b0, Wself1, Wneigh1, b1,
           Wd1, bd1, Wd2, bd2, Wd3, bd3):
    zfeat = jnp.zeros((RPT, D), jnp.float32)
    zdeg = jnp.zeros((N_PAD,), jnp.float32)

    eshape = (NW, NSB, SB, CH)
    src0 = edge_index_l0[0].reshape(eshape)
    dst0 = edge_index_l0[1].reshape(eshape)
    src1 = edge_index_l1[0].reshape(eshape)
    dst1 = edge_index_l1[1].reshape(eshape)

    aggp0, degp0 = _seg_sum(x, src0, dst0, zfeat, zdeg)
    h1 = _combine_relu(x, aggp0, degp0, Wself0.T, Wneigh0.T, b0.reshape(1, D))
    aggp1, degp1 = _seg_sum(h1, src1, dst1, zfeat, zdeg)
    h2 = _combine_lin(h1, aggp1, degp1, Wself1.T, Wneigh1.T, b1.reshape(1, D))
    def _pad_idx(v):
        return jnp.concatenate(
            [v, jnp.zeros((NPPAD * PCH - N_PAIR,), v.dtype)]).reshape(NPPAD // NW * NW, PCH)

    a, b = _pair_gather(h2, _pad_idx(pos_edge_index[0]), _pad_idx(pos_edge_index[1]),
                        _pad_idx(neg_edge_index[0]), _pad_idx(neg_edge_index[1]))
    out = _mlp(a, b, Wd1.T, bd1.reshape(1, D), Wd2.T, bd2.reshape(1, D),
               Wd3.T, bd3.reshape(1, 1))
    return out[:N_PAIR], out[N_PAIR:]
